# Initial kernel scaffold; baseline (speedup 1.0000x reference)
import jax
import jax.numpy as jnp
from jax.experimental import pallas as pl


def kernel(z, atom_types, inj_W, inj_b, edge_W0, edge_b0, edge_W1, edge_b1, node_W0, node_b0, node_W1, node_b1, coord_W0, coord_b0, coord_W1):
    raise NotImplementedError("write your pallas kernel here")


# speedup vs baseline: 30.9986x; 30.9986x over previous
import jax
import jax.numpy as jnp
from jax.experimental import pallas as pl
from jax.experimental.pallas import tpu as pltpu

_B = 128
_N = 64
_F = 128
_L = 4


def _egnn_body(z_ref, at_ref, injb_ref, Wia_ref, Wiz_ref,
               W0a_ref, W0b_ref, w0c_ref, b0_ref,
               W1_ref, b1_ref,
               Wnh_ref, Wna_ref, nb0_ref, nW1_ref, nb1_ref,
               Wc_ref, bc_ref, w1c_ref,
               out_ref):
    n, f = _N, _F
    at = at_ref[...]
    zrow = z_ref[0]
    h = at @ Wia_ref[...] + zrow @ Wiz_ref[...] + injb_ref[...]
    coords = jnp.zeros((n, f), dtype=jnp.float32)

    for l in range(_L):
        rel3 = coords[:, None, :] - coords[None, :, :]
        dsq3 = jnp.sum(rel3 * rel3, axis=2, keepdims=True)

        A = h @ W0a_ref[l]
        Bc = h @ W0b_ref[l] + b0_ref[l][None, :]
        pre3 = A[:, None, :] + Bc[None, :, :] + dsq3 * w0c_ref[l][None, None, :]
        r3 = jnp.maximum(pre3, 0.0)

        S = jnp.sum(r3, axis=1) - jnp.maximum(A + Bc, 0.0)
        agg = S @ W1_ref[l] + float(n - 1) * b1_ref[l][None, :]
        hn = (jnp.maximum(h @ Wnh_ref[l] + agg @ Wna_ref[l]
                          + nb0_ref[l][None, :], 0.0)
              @ nW1_ref[l] + nb1_ref[l][None, :])

        r2 = r3.reshape(n * n, f)
        t = jnp.maximum(r2 @ Wc_ref[l] + bc_ref[l][None, :], 0.0)
        cw = jnp.sum(t * w1c_ref[l][None, :], axis=1, keepdims=True)
        upd = (cw * rel3.reshape(n * n, f)).reshape(n, n, f)
        coords = coords + jnp.sum(upd, axis=1)
        h = hn

    out_ref[0] = coords


def kernel(z, atom_types, inj_W, inj_b, edge_W0, edge_b0, edge_W1, edge_b1,
           node_W0, node_b0, node_W1, node_b1, coord_W0, coord_b0, coord_W1):
    f = _F
    Wia = inj_W[:f]
    Wiz = inj_W[f:]
    W0a = edge_W0[:, :f, :]
    W0b = edge_W0[:, f:2 * f, :]
    w0c = edge_W0[:, 2 * f, :]
    Wnh = node_W0[:, :f, :]
    Wna = node_W0[:, f:, :]
    Wc = jnp.einsum("lij,ljk->lik", edge_W1, coord_W0)
    bc = jnp.einsum("lj,ljk->lk", edge_b1, coord_W0) + coord_b0
    w1c = coord_W1[:, :, 0]

    full = lambda a: pl.BlockSpec(a.shape, lambda b: (0,) * a.ndim)
    injb2 = inj_b.reshape(1, f)
    z3 = z.reshape(_B, 1, z.shape[1])

    out = pl.pallas_call(
        _egnn_body,
        grid=(_B,),
        in_specs=[
            pl.BlockSpec((1, 1, z.shape[1]), lambda b: (b, 0, 0)),
            pl.BlockSpec((_N, f), lambda b: (b, 0)),
            full(injb2), full(Wia), full(Wiz),
            full(W0a), full(W0b), full(w0c), full(edge_b0),
            full(edge_W1), full(edge_b1),
            full(Wnh), full(Wna), full(node_b0), full(node_W1), full(node_b1),
            full(Wc), full(bc), full(w1c),
        ],
        out_specs=pl.BlockSpec((1, _N, f), lambda b: (b, 0, 0)),
        out_shape=jax.ShapeDtypeStruct((_B, _N, f), jnp.float32),
        compiler_params=pltpu.CompilerParams(
            dimension_semantics=("arbitrary",),
        ),
    )(z3, atom_types, injb2, Wia, Wiz, W0a, W0b, w0c, edge_b0,
      edge_W1, edge_b1, Wnh, Wna, node_b0, node_W1, node_b1, Wc, bc, w1c)
    return out[:, :, :3]
